# baseline (speedup 1.0000x reference)
import jax
import jax.numpy as jnp
from jax.experimental import pallas as pl
from jax.experimental.pallas import tpu as pltpu

_NSPLIT = 2


def _roll_lanes(p, s):
    L = p.shape[-1]
    k = s % L
    if k == 0:
        return p
    return jnp.concatenate([p[:, k:], p[:, :k]], axis=-1)


def _make_body(H, W, cmid, block_n):
    L = H * W

    def _body(*refs):
        (w1_ref, g_ref, b_ref, m_ref, v_ref, w2_ref, e_ref, b2_ref,
         o_ref) = refs[_NSPLIT:]
        x_refs = refs[:_NSPLIT]
        eps = e_ref[0]
        b2 = b2_ref[0]

        scale_row = g_ref[...] / jnp.sqrt(v_ref[...] + eps)
        shift = jnp.swapaxes(b_ref[...] - m_ref[...] * scale_row,
                             0, 1)
        w2 = jnp.swapaxes(w2_ref[...], 0, 1)
        w1 = w1_ref[...]
        rows = []
        for ky in range(3):
            blk = [jnp.swapaxes(w1[ky, kx] * scale_row, 0, 1)
                   for kx in range(3)]
            rows.append(jnp.concatenate(blk, axis=1))
        a = jnp.concatenate(rows, axis=0).astype(jnp.bfloat16)

        lane = jax.lax.broadcasted_iota(jnp.int32, (1, L), 1)
        col = lane % W
        not_last_col = col != (W - 1)
        not_first_col = col != 0
        below_last_row = lane < (L - W)
        above_first_row = lane >= W

        zero_b = jnp.zeros((), jnp.bfloat16)
        zero_f = jnp.zeros((), jnp.float32)
        cq = x_refs[0].shape[1]
        for i in range(block_n):
            parts = [
                jnp.reshape(r[i], (cq, L)).astype(jnp.bfloat16)
                for r in x_refs
            ]
            xi = jnp.concatenate(parts, axis=0)
            x_p1 = jnp.where(not_last_col, _roll_lanes(xi, 1), zero_b)
            x_m1 = jnp.where(not_first_col, _roll_lanes(xi, -1), zero_b)
            xs = jnp.concatenate([x_m1, xi, x_p1], axis=0)

            y3 = jax.lax.dot_general(
                a, xs, (((1,), (0,)), ((), ())),
                preferred_element_type=jnp.float32)

            acc = y3[cmid:2 * cmid]
            acc = acc + jnp.where(below_last_row,
                                  _roll_lanes(y3[2 * cmid:], W), zero_f)
            acc = acc + jnp.where(above_first_row,
                                  _roll_lanes(y3[:cmid], -W), zero_f)

            y = jnp.maximum(acc + shift, 0.0)
            row = jnp.sum(y * w2, axis=0, keepdims=True) + b2
            o_ref[pl.ds(i, 1), :] = row

    return _body


def _forward(x, w1, gamma, beta, run_mean, run_var, eps, w2, b2,
             block_n=8, interpret=False):
    N, Cin, H, W = x.shape
    Cmid = w1.shape[3]
    L = H * W
    cq = Cin // _NSPLIT

    grid = (N // block_n,)

    def xspec(k):
        return pl.BlockSpec((block_n, cq, H, W), lambda n, k=k: (n, k, 0, 0))

    def vrow():
        return pl.BlockSpec((1, Cmid), lambda n: (0, 0))

    out_flat = pl.pallas_call(
        _make_body(H, W, Cmid, block_n),
        out_shape=jax.ShapeDtypeStruct((N, L), jnp.float32),
        grid=grid,
        in_specs=[*(xspec(k) for k in range(_NSPLIT)),
                  pl.BlockSpec((3, 3, Cin, Cmid), lambda n: (0, 0, 0, 0)),
                  vrow(), vrow(), vrow(), vrow(), vrow(),
                  pl.BlockSpec(memory_space=pltpu.MemorySpace.SMEM),
                  pl.BlockSpec(memory_space=pltpu.MemorySpace.SMEM)],
        out_specs=pl.BlockSpec((block_n, L), lambda n: (n, 0)),
        compiler_params=pltpu.CompilerParams(
            dimension_semantics=("parallel",),
            vmem_limit_bytes=100 * 1024 * 1024),
        interpret=interpret,
    )(*([x] * _NSPLIT),
      w1.astype(jnp.float32),
      gamma.reshape(1, Cmid).astype(jnp.float32),
      beta.reshape(1, Cmid).astype(jnp.float32),
      run_mean.reshape(1, Cmid).astype(jnp.float32),
      run_var.reshape(1, Cmid).astype(jnp.float32),
      w2.reshape(1, Cmid).astype(jnp.float32),
      eps.reshape(1).astype(jnp.float32),
      b2.reshape(1).astype(jnp.float32))

    return out_flat.reshape(N, 1, H, W)


def kernel(x, w1, gamma, beta, run_mean, run_var, eps, w2, b2):
    return _forward(x, w1, gamma, beta, run_mean, run_var, eps, w2, b2)


# speedup vs baseline: 1.0077x; 1.0077x over previous
import jax
import jax.numpy as jnp
from jax.experimental import pallas as pl
from jax.experimental.pallas import tpu as pltpu

_NSPLIT = 4


def _roll_lanes(p, s):
    L = p.shape[-1]
    k = s % L
    if k == 0:
        return p
    return jnp.concatenate([p[:, k:], p[:, :k]], axis=-1)


def _make_body(H, W, cmid, block_n):
    L = H * W

    def _body(*refs):
        (w1_ref, g_ref, b_ref, m_ref, v_ref, w2_ref, e_ref, b2_ref,
         o_ref) = refs[_NSPLIT:]
        x_refs = refs[:_NSPLIT]
        eps = e_ref[0]
        b2 = b2_ref[0]

        scale_row = g_ref[...] / jnp.sqrt(v_ref[...] + eps)
        shift = jnp.swapaxes(b_ref[...] - m_ref[...] * scale_row,
                             0, 1)
        w2 = jnp.swapaxes(w2_ref[...], 0, 1)
        w1 = w1_ref[...]
        rows = []
        for ky in range(3):
            blk = [jnp.swapaxes(w1[ky, kx] * scale_row, 0, 1)
                   for kx in range(3)]
            rows.append(jnp.concatenate(blk, axis=1))
        a = jnp.concatenate(rows, axis=0).astype(jnp.bfloat16)

        lane = jax.lax.broadcasted_iota(jnp.int32, (1, L), 1)
        col = lane % W
        not_last_col = col != (W - 1)
        not_first_col = col != 0
        below_last_row = lane < (L - W)
        above_first_row = lane >= W

        zero_b = jnp.zeros((), jnp.bfloat16)
        zero_f = jnp.zeros((), jnp.float32)
        cq = x_refs[0].shape[1]
        for i in range(block_n):
            parts = [
                jnp.reshape(r[i], (cq, L)).astype(jnp.bfloat16)
                for r in x_refs
            ]
            xi = jnp.concatenate(parts, axis=0)
            x_p1 = jnp.where(not_last_col, _roll_lanes(xi, 1), zero_b)
            x_m1 = jnp.where(not_first_col, _roll_lanes(xi, -1), zero_b)
            xs = jnp.concatenate([x_m1, xi, x_p1], axis=0)

            y3 = jax.lax.dot_general(
                a, xs, (((1,), (0,)), ((), ())),
                preferred_element_type=jnp.float32)

            acc = y3[cmid:2 * cmid]
            acc = acc + jnp.where(below_last_row,
                                  _roll_lanes(y3[2 * cmid:], W), zero_f)
            acc = acc + jnp.where(above_first_row,
                                  _roll_lanes(y3[:cmid], -W), zero_f)

            y = jnp.maximum(acc + shift, 0.0)
            row = jnp.sum(y * w2, axis=0, keepdims=True) + b2
            o_ref[pl.ds(i, 1), :] = row

    return _body


def _forward(x, w1, gamma, beta, run_mean, run_var, eps, w2, b2,
             block_n=8, interpret=False):
    N, Cin, H, W = x.shape
    Cmid = w1.shape[3]
    L = H * W
    cq = Cin // _NSPLIT

    grid = (N // block_n,)

    def xspec(k):
        return pl.BlockSpec((block_n, cq, H, W), lambda n, k=k: (n, k, 0, 0))

    def vrow():
        return pl.BlockSpec((1, Cmid), lambda n: (0, 0))

    out_flat = pl.pallas_call(
        _make_body(H, W, Cmid, block_n),
        out_shape=jax.ShapeDtypeStruct((N, L), jnp.float32),
        grid=grid,
        in_specs=[*(xspec(k) for k in range(_NSPLIT)),
                  pl.BlockSpec((3, 3, Cin, Cmid), lambda n: (0, 0, 0, 0)),
                  vrow(), vrow(), vrow(), vrow(), vrow(),
                  pl.BlockSpec(memory_space=pltpu.MemorySpace.SMEM),
                  pl.BlockSpec(memory_space=pltpu.MemorySpace.SMEM)],
        out_specs=pl.BlockSpec((block_n, L), lambda n: (n, 0)),
        compiler_params=pltpu.CompilerParams(
            dimension_semantics=("parallel",),
            vmem_limit_bytes=60 * 1024 * 1024),
        interpret=interpret,
    )(*([x] * _NSPLIT),
      w1.astype(jnp.float32),
      gamma.reshape(1, Cmid).astype(jnp.float32),
      beta.reshape(1, Cmid).astype(jnp.float32),
      run_mean.reshape(1, Cmid).astype(jnp.float32),
      run_var.reshape(1, Cmid).astype(jnp.float32),
      w2.reshape(1, Cmid).astype(jnp.float32),
      eps.reshape(1).astype(jnp.float32),
      b2.reshape(1).astype(jnp.float32))

    return out_flat.reshape(N, 1, H, W)


def kernel(x, w1, gamma, beta, run_mean, run_var, eps, w2, b2):
    return _forward(x, w1, gamma, beta, run_mean, run_var, eps, w2, b2)
